# baseline (speedup 1.0000x reference)
import jax
import jax.numpy as jnp
from jax import lax
from jax.experimental import pallas as pl
from jax.experimental.pallas import tpu as pltpu
from jax.experimental.pallas import tpu_sc as plsc

NUM_POSITIONS = 8192
EMBED_DIM = 1024
B_TOTAL = 4 * 8192

_info = plsc.get_sparse_core_info()
_NC, _NS = _info.num_cores, _info.num_subcores
_NW = _NC * _NS
_B_PER_W = B_TOTAL // _NW
_CHUNK = 16
_GRP = 2
_NPAIR = 3
_N_PAIRS = _B_PER_W // (_CHUNK * _GRP)


def _gather_kernel(x_hbm, w_hbm, out_hbm, idx_v, bufs_v, gsems, wsems):
    wid = lax.axis_index("s") * _NC + lax.axis_index("c")
    base = wid * _B_PER_W
    pltpu.sync_copy(x_hbm.at[pl.ds(base, _B_PER_W)], idx_v)

    def g_start(p, h):
        i = p * _GRP + h
        pltpu.async_copy(w_hbm.at[idx_v.at[pl.ds(i * _CHUNK, _CHUNK)]],
                         bufs_v.at[p % _NPAIR].at[pl.ds(h * _CHUNK, _CHUNK)],
                         gsems.at[(p % _NPAIR) * _GRP + h])

    def g_wait(p, h):
        pltpu.make_async_copy(w_hbm.at[pl.ds(0, _CHUNK)],
                              bufs_v.at[p % _NPAIR].at[pl.ds(h * _CHUNK, _CHUNK)],
                              gsems.at[(p % _NPAIR) * _GRP + h]).wait()

    def w_start(p):
        pltpu.async_copy(bufs_v.at[p % _NPAIR],
                         out_hbm.at[pl.ds(base + p * _GRP * _CHUNK,
                                          _GRP * _CHUNK)],
                         wsems.at[p % _NPAIR])

    def w_wait(p):
        pltpu.make_async_copy(bufs_v.at[p % _NPAIR],
                              out_hbm.at[pl.ds(base, _GRP * _CHUNK)],
                              wsems.at[p % _NPAIR]).wait()

    def g_start_pair(p):
        for h in range(_GRP):
            g_start(p, h)

    g_start_pair(0)
    g_start_pair(1)
    g_wait(0, 0)
    g_wait(0, 1)
    w_start(0)
    g_start_pair(2)
    g_wait(1, 0)
    g_wait(1, 1)
    w_start(1)
    w_wait(0)
    g_start_pair(3)

    def body(j, _):
        for s in range(3):
            p = 2 + 3 * j + s
            g_wait(p, 0)
            g_wait(p, 1)
            w_start(p)
            w_wait(p + 2)
            g_start_pair(p + 2)
        return ()

    lax.fori_loop(0, 9, body, (), unroll=False)

    g_wait(29, 0)
    g_wait(29, 1)
    w_start(29)
    w_wait(28)
    g_start_pair(31)
    g_wait(30, 0)
    g_wait(30, 1)
    w_start(30)
    w_wait(29)
    g_wait(31, 0)
    g_wait(31, 1)
    w_start(31)
    w_wait(30)
    w_wait(31)


@jax.jit
def kernel(x, weight):
    x_flat = x.reshape(B_TOTAL).astype(jnp.int32)
    mesh = plsc.VectorSubcoreMesh(core_axis_name="c", subcore_axis_name="s")
    out = pl.kernel(
        _gather_kernel,
        mesh=mesh,
        out_type=jax.ShapeDtypeStruct((B_TOTAL, EMBED_DIM), jnp.float32),
        scratch_types=[
            pltpu.VMEM((_B_PER_W,), jnp.int32),
            pltpu.VMEM((_NPAIR, _GRP * _CHUNK, EMBED_DIM), jnp.float32),
            pltpu.SemaphoreType.DMA((_NPAIR * _GRP,)),
            pltpu.SemaphoreType.DMA((_NPAIR,)),
        ],
    )(x_flat, weight)
    return out.reshape(x.shape[0], x.shape[1], EMBED_DIM)
